# Initial kernel scaffold; baseline (speedup 1.0000x reference)
import jax
import jax.numpy as jnp
from jax.experimental import pallas as pl


def kernel(inputs, word_table, pos_table):
    raise NotImplementedError("write your pallas kernel here")


# speedup vs baseline: 1.3478x; 1.3478x over previous
import functools

import jax
import jax.numpy as jnp
from jax import lax
from jax.experimental import pallas as pl
from jax.experimental.pallas import tpu as pltpu
from jax.experimental.pallas import tpu_sc as plsc

VOCAB = 1000000
SEQ = 200
DIM = 32
BATCH = 4096
LANES = 16

NC = 2
NS = 16
NW = NC * NS
ROWS_PER_W = BATCH // NW
HALF = SEQ // 2
NBUF = 8
LOOK = 4


def _body(idx_hbm, word_hbm, pos_hbm, out_hbm, idx_v, pos_v, rows_v,
          gsem, ssem):
    w = lax.axis_index("s") * NC + lax.axis_index("c")
    pltpu.sync_copy(idx_hbm.at[w], idx_v)
    pltpu.sync_copy(pos_hbm, pos_v)
    base = w * (ROWS_PER_W * SEQ)

    def issue_gather(j, s):
        pltpu.async_copy(
            word_hbm.at[idx_v.at[2 * j]],
            rows_v.at[s].at[pl.ds(0, HALF)], gsem.at[s])
        pltpu.async_copy(
            word_hbm.at[idx_v.at[2 * j + 1]],
            rows_v.at[s].at[pl.ds(HALF, HALF)], gsem.at[s])

    for b in range(LOOK):
        issue_gather(b, b)

    def it(i, carry):
        b = lax.rem(i, NBUF)
        bj = lax.rem(i + LOOK, NBUF)

        pltpu.make_async_copy(
            out_hbm.at[pl.ds(0, SEQ)], rows_v.at[b], gsem.at[b]).wait()

        @pl.when(i + LOOK < ROWS_PER_W)
        def _():
            @pl.when(i >= NBUF - LOOK)
            def _():
                pltpu.make_async_copy(
                    rows_v.at[bj], out_hbm.at[pl.ds(0, SEQ)],
                    ssem.at[bj]).wait()
            issue_gather(i + LOOK, bj)

        for r in range(SEQ):
            for j in range(DIM // LANES):
                plsc.addupdate(
                    rows_v.at[b].at[r, pl.ds(j * LANES, LANES)],
                    pos_v[r, pl.ds(j * LANES, LANES)],
                )

        pltpu.async_copy(
            rows_v.at[b], out_hbm.at[pl.ds(base + i * SEQ, SEQ)],
            ssem.at[b])
        return carry

    lax.fori_loop(0, ROWS_PER_W, it, 0)

    for b in range(NBUF):
        pltpu.make_async_copy(
            rows_v.at[b], out_hbm.at[pl.ds(0, SEQ)], ssem.at[b]).wait()


_grid_kernel = functools.partial(
    pl.kernel,
    mesh=plsc.VectorSubcoreMesh(core_axis_name="c", subcore_axis_name="s"),
    out_type=jax.ShapeDtypeStruct((BATCH * SEQ, DIM), jnp.float32),
    compiler_params=pltpu.CompilerParams(use_tc_tiling_on_sc=False),
    scratch_types=[
        pltpu.VMEM((2 * ROWS_PER_W, HALF), jnp.int32),
        pltpu.VMEM((SEQ, DIM), jnp.float32),
        pltpu.VMEM((NBUF, SEQ, DIM), jnp.float32),
        pltpu.SemaphoreType.DMA((NBUF,)),
        pltpu.SemaphoreType.DMA((NBUF,)),
    ],
)(_body)


def kernel(inputs, word_table, pos_table):
    idx3 = inputs.reshape(NW, 2 * ROWS_PER_W, HALF)
    out = _grid_kernel(idx3, word_table, pos_table)
    return out.reshape(BATCH, SEQ, DIM)


# speedup vs baseline: 1.3518x; 1.0030x over previous
import functools

import jax
import jax.numpy as jnp
from jax import lax
from jax.experimental import pallas as pl
from jax.experimental.pallas import tpu as pltpu
from jax.experimental.pallas import tpu_sc as plsc

VOCAB = 1000000
SEQ = 200
DIM = 32
BATCH = 4096
LANES = 16

NC = 2
NS = 16
NW = NC * NS
ROWS_PER_W = BATCH // NW
H0 = 104
H1 = 96
NBUF = 8
LOOK = 4


def _body(idx_hbm, word_hbm, pos_hbm, out_hbm, idx_v, pos_v, rows_v,
          gsem, ssem):
    w = lax.axis_index("s") * NC + lax.axis_index("c")
    pltpu.sync_copy(idx_hbm.at[pl.ds(w * ROWS_PER_W, ROWS_PER_W)], idx_v)
    pltpu.sync_copy(pos_hbm, pos_v)
    row0 = w * ROWS_PER_W

    def issue_gather(j, s):
        pltpu.async_copy(
            word_hbm.at[idx_v.at[j, pl.ds(0, H0)]],
            rows_v.at[s].at[pl.ds(0, H0)], gsem.at[s])
        pltpu.async_copy(
            word_hbm.at[idx_v.at[j, pl.ds(H0, H1)]],
            rows_v.at[s].at[pl.ds(H0, H1)], gsem.at[s])

    for b in range(LOOK):
        issue_gather(b, b)

    def it(i, carry):
        b = lax.rem(i, NBUF)
        bj = lax.rem(i + LOOK, NBUF)

        pltpu.make_async_copy(
            out_hbm.at[0], rows_v.at[b], gsem.at[b]).wait()

        @pl.when(i + LOOK < ROWS_PER_W)
        def _():
            @pl.when(i >= NBUF - LOOK)
            def _():
                pltpu.make_async_copy(
                    rows_v.at[bj], out_hbm.at[0], ssem.at[bj]).wait()
            issue_gather(i + LOOK, bj)

        for r in range(SEQ):
            for j in range(DIM // LANES):
                plsc.addupdate(
                    rows_v.at[b].at[r, pl.ds(j * LANES, LANES)],
                    pos_v[r, pl.ds(j * LANES, LANES)],
                )

        pltpu.async_copy(rows_v.at[b], out_hbm.at[row0 + i], ssem.at[b])
        return carry

    lax.fori_loop(0, ROWS_PER_W, it, 0)

    for b in range(NBUF):
        pltpu.make_async_copy(
            rows_v.at[b], out_hbm.at[0], ssem.at[b]).wait()


_grid_kernel = functools.partial(
    pl.kernel,
    mesh=plsc.VectorSubcoreMesh(core_axis_name="c", subcore_axis_name="s"),
    out_type=jax.ShapeDtypeStruct((BATCH, SEQ, DIM), jnp.float32),
    compiler_params=pltpu.CompilerParams(use_tc_tiling_on_sc=False),
    scratch_types=[
        pltpu.VMEM((ROWS_PER_W, SEQ), jnp.int32),
        pltpu.VMEM((SEQ, DIM), jnp.float32),
        pltpu.VMEM((NBUF, SEQ, DIM), jnp.float32),
        pltpu.SemaphoreType.DMA((NBUF,)),
        pltpu.SemaphoreType.DMA((NBUF,)),
    ],
)(_body)


def kernel(inputs, word_table, pos_table):
    return _grid_kernel(inputs, word_table, pos_table)
